# baseline (speedup 1.0000x reference)
import functools

import jax
import jax.numpy as jnp
from jax import lax
from jax.experimental import pallas as pl
from jax.experimental.pallas import tpu as pltpu
from jax.experimental.pallas import tpu_sc as plsc

N = 50000
D = 128
R = 0.2
K = 64
C = 12500
CHUNK = 250
H1 = 128
H2 = 128
EPS = 1e-5

_G = (N + 1023) // 1024
_NPAD = _G * 1024
_CROWS = (C + 127) // 128
_CPAD = _CROWS * 128
_BIG_I32 = 2**30


def _fps_body(pxyz_ref, out_ref, dists_ref):
    flat3 = (
        jax.lax.broadcasted_iota(jnp.int32, (_G, 8, 128), 0) * 1024
        + jax.lax.broadcasted_iota(jnp.int32, (_G, 8, 128), 1) * 128
        + jax.lax.broadcasted_iota(jnp.int32, (_G, 8, 128), 2)
    )
    dists_ref[...] = jnp.where(flat3 < N, jnp.inf, -jnp.inf).astype(jnp.float32)

    def body(i, carry):
        last, rowv = carry
        g = last // 1024
        s = (last // 128) % 8
        l = last % 128
        tile = pxyz_ref[:, pl.ds(g, 1)]
        s_io = jax.lax.broadcasted_iota(jnp.int32, (1, 8, 128), 1)
        l_io = jax.lax.broadcasted_iota(jnp.int32, (1, 8, 128), 2)
        sel = jnp.logical_and(s_io == s, l_io == l)
        lx = jnp.sum(jnp.where(sel, tile[0], 0.0))
        ly = jnp.sum(jnp.where(sel, tile[1], 0.0))
        lz = jnp.sum(jnp.where(sel, tile[2], 0.0))
        vmax = jnp.full((1, 8, 128), -jnp.inf, jnp.float32)
        vpl = jnp.zeros((1, 8, 128), jnp.int32)
        for gg in range(_G):
            dx = pxyz_ref[0, pl.ds(gg, 1)] - lx
            dy = pxyz_ref[1, pl.ds(gg, 1)] - ly
            dz = pxyz_ref[2, pl.ds(gg, 1)] - lz
            d = (dx * dx + dz * dz) + dy * dy
            nd = jnp.minimum(dists_ref[pl.ds(gg, 1)], d)
            dists_ref[pl.ds(gg, 1)] = nd
            upd = nd > vmax
            vmax = jnp.where(upd, nd, vmax)
            vpl = jnp.where(upd, gg, vpl)
        m = jnp.max(vmax)
        slot_flat = s_io * 128 + l_io
        nxt = jnp.min(jnp.where(vmax == m, vpl * 1024 + slot_flat, _BIG_I32))
        lane = i % 128
        rowv = jnp.where(lax.broadcasted_iota(jnp.int32, (1, 128), 1) == lane,
                         nxt, rowv)

        @pl.when(lane == 127)
        def _():
            out_ref[pl.ds(i // 128, 1), :] = rowv

        return nxt, rowv

    _, rowv = jax.lax.fori_loop(
        1, C, body, (jnp.int32(0), jnp.zeros((1, 128), jnp.int32)))
    out_ref[pl.ds(_CROWS - 1, 1), :] = rowv


def _fps_pallas(pos):
    pad = jnp.full((_NPAD - N, 3), 30.0, jnp.float32)
    pxyz = jnp.concatenate([pos, pad], axis=0).T.reshape(3, _G, 8, 128)
    out = pl.pallas_call(
        _fps_body,
        out_shape=jax.ShapeDtypeStruct((_CROWS, 128), jnp.int32),
        scratch_shapes=[pltpu.VMEM((_G, 8, 128), jnp.float32)],
    )(pxyz)
    return out.reshape(_CPAD)[:C]


_NSC = 32
_CPER = 392
_CPAD2 = _NSC * _CPER
_PCH = 12544
_NPAD3 = 4 * _PCH
_GC = 8
_CAPC = 4096
_NBB = 1024
import numpy as _np

_RR = float(_np.float32(0.2) * _np.float32(0.2))
_INF = float("inf")


def _ilist(x):
    return [x]


def _bq_body(px_h, py_h, pz_h, cx_h, cy_h, cz_h, oidx_h, ocnt_h,
             pxb, pyb, pzb, cbx, cby, cbz, cand_d, cand_i, cntb,
             hist, cum, bbd, bbi, orow, ocb):
    wid = lax.axis_index("s") * 2 + lax.axis_index("c")
    cbase = wid * _CPER
    pltpu.sync_copy(cx_h.at[pl.ds(cbase, _CPER)], cbx)
    pltpu.sync_copy(cy_h.at[pl.ds(cbase, _CPER)], cby)
    pltpu.sync_copy(cz_h.at[pl.ds(cbase, _CPER)], cbz)
    l16 = lax.iota(jnp.int32, 16)
    lane0 = l16 == 0
    zero16 = jnp.zeros((16,), jnp.int32)

    def splat_load(ref, i):
        return plsc.load_gather(ref, _ilist(zero16 + i))

    def group_loop(g, _):
        cntb[pl.ds(0, 16)] = zero16

        def chunk_loop(ch, _):
            off = ch * _PCH
            pltpu.sync_copy(px_h.at[pl.ds(off, _PCH)], pxb)
            pltpu.sync_copy(py_h.at[pl.ds(off, _PCH)], pyb)
            pltpu.sync_copy(pz_h.at[pl.ds(off, _PCH)], pzb)

            def cent_loop(cc, _):
                cl = g * _GC + cc
                cxv = splat_load(cbx, cl)
                cyv = splat_load(cby, cl)
                czv = splat_load(cbz, cl)
                cnt0 = splat_load(cntb, cc)

                def vec_loop(j, cnt):
                    ds_ = []
                    ms_ = []
                    pcs = []
                    for u in range(8):
                        b = j * 128 + u * 16
                        dx = pxb[pl.ds(b, 16)] - cxv
                        dy = pyb[pl.ds(b, 16)] - cyv
                        dz = pzb[pl.ds(b, 16)] - czv
                        d = (dx * dx + dz * dz) + dy * dy
                        m = d <= _RR
                        ds_.append(d)
                        ms_.append(m)
                        pcs.append(plsc.cumsum(m.astype(jnp.int32)))
                    for u in range(8):
                        b = j * 128 + u * 16
                        pos = jnp.minimum(cnt + pcs[u] - 1, _CAPC - 1) + cc * _CAPC
                        pid = off + b + l16
                        plsc.store_scatter(cand_d, _ilist(pos), ds_[u], mask=ms_[u])
                        plsc.store_scatter(cand_i, _ilist(pos), pid, mask=ms_[u])
                        cnt = cnt + plsc.all_reduce_population_count(ms_[u])
                    return cnt

                cnt = lax.fori_loop(0, _PCH // 128, vec_loop, cnt0)
                plsc.store_scatter(cntb, _ilist(zero16 + cc), cnt,
                                   mask=lane0)
                return 0

            lax.fori_loop(0, _GC, cent_loop, 0)
            return 0

        lax.fori_loop(0, 4, chunk_loop, 0)

        def select_loop(cc, _):
            cl = g * _GC + cc
            total = jnp.minimum(jnp.max(splat_load(cntb, cc)), _CAPC)
            nv = (total + 15) // 16
            cb = cc * _CAPC

            def hclr(k, _):
                hist[pl.ds(k * 16, 16)] = zero16
                return 0

            lax.fori_loop(0, 16, hclr, 0)

            def hsweep(j, _):
                b = cb + j * 16
                lm = (j * 16 + l16) < total
                d = jnp.where(lm, cand_d[pl.ds(b, 16)], 0.0)
                bk = jnp.minimum((d * 6400.0).astype(jnp.int32), 255)
                plsc.addupdate_scatter(hist, _ilist(bk),
                                       jnp.where(lm, 1, 0), mask=lm)
                return 0

            lax.fori_loop(0, nv, hsweep, 0)

            def csum(k, carry):
                s = plsc.cumsum(hist[pl.ds(k * 16, 16)]) + carry
                cum[pl.ds(k * 16, 16)] = s
                return jnp.max(s)

            lax.fori_loop(0, 16, csum, jnp.int32(0))

            def bstar_loop(k, carry):
                bs, lo = carry
                cv = cum[pl.ds(k * 16, 16)]
                bio = k * 16 + l16
                bs = jnp.minimum(bs, jnp.min(jnp.where(cv >= K, bio, 1024)))
                lo = jnp.maximum(lo, jnp.max(jnp.where(cv < K, cv, 0)))
                return bs, lo

            bstar, lo = lax.fori_loop(0, 16, bstar_loop,
                                      (jnp.int32(1024), jnp.int32(0)))
            need = jnp.where(bstar < 1024, K - lo, 0)

            def oclr(k, _):
                orow[pl.ds(k * 16, 16)] = zero16
                return 0

            lax.fori_loop(0, K // 16, oclr, 0)

            def ssweep(j, carry):
                osel, bbc = carry
                b = cb + j * 16
                lm = (j * 16 + l16) < total
                d = jnp.where(lm, cand_d[pl.ds(b, 16)], _INF)
                pid = cand_i[pl.ds(b, 16)]
                bk = jnp.minimum((d * 6400.0).astype(jnp.int32), 255)
                sm = lm & (bk < bstar)
                pc = plsc.cumsum(sm.astype(jnp.int32))
                pos = jnp.minimum(osel + pc - 1, K - 1)
                plsc.store_scatter(orow, _ilist(pos), pid, mask=sm)
                osel = osel + plsc.all_reduce_population_count(sm)
                bm = lm & (bk == bstar)
                pc2 = plsc.cumsum(bm.astype(jnp.int32))
                pos2 = jnp.minimum(bbc + pc2 - 1, _NBB - 1)
                plsc.store_scatter(bbd, _ilist(pos2), d, mask=bm)
                plsc.store_scatter(bbi, _ilist(pos2), pid, mask=bm)
                bbc = bbc + plsc.all_reduce_population_count(bm)
                return osel, bbc

            osel, bbc = lax.fori_loop(0, nv, ssweep, (zero16, zero16))
            nb = jnp.minimum(jnp.max(bbc), _NBB)
            nbv = (nb + 15) // 16

            def extract(_, osel):
                def minp(j, ms):
                    lm = (j * 16 + l16) < nb
                    v = jnp.where(lm, bbd[pl.ds(j * 16, 16)], _INF)
                    return jnp.minimum(ms, jnp.min(v))

                ms = lax.fori_loop(0, nbv, minp, jnp.float32(_INF))

                def idxp(j, ix):
                    lm = (j * 16 + l16) < nb
                    v = bbd[pl.ds(j * 16, 16)]
                    pid = bbi[pl.ds(j * 16, 16)]
                    sel = lm & (v == ms)
                    return jnp.minimum(ix, jnp.min(jnp.where(sel, pid, _BIG_I32)))

                ix = lax.fori_loop(0, nbv, idxp, jnp.int32(_BIG_I32))

                def markp(j, _):
                    b = j * 16
                    v = bbd[pl.ds(b, 16)]
                    pid = bbi[pl.ds(b, 16)]
                    hit = (v == ms) & (pid == ix)
                    bbd[pl.ds(b, 16)] = jnp.where(hit, _INF, v)
                    return 0

                lax.fori_loop(0, nbv, markp, 0)
                plsc.store_scatter(orow, _ilist(jnp.minimum(osel, K - 1)),
                                   zero16 + ix, mask=lane0)
                return osel + 1

            osel = lax.fori_loop(0, need, extract, osel)
            pltpu.sync_copy(orow, oidx_h.at[pl.ds((cbase + cl) * K, K)])
            plsc.store_scatter(ocb, _ilist(zero16 + cl),
                               jnp.minimum(osel, K), mask=lane0)
            return 0

        lax.fori_loop(0, _GC, select_loop, 0)
        return 0

    lax.fori_loop(0, _CPER // _GC, group_loop, 0)
    pltpu.sync_copy(ocb, ocnt_h.at[pl.ds(cbase, _CPER)])


def _ball_query_flat(pos, cpos):
    ppad = jnp.full((_NPAD3 - N,), 30.0, jnp.float32)
    px = jnp.concatenate([pos[:, 0], ppad])
    py = jnp.concatenate([pos[:, 1], ppad])
    pz = jnp.concatenate([pos[:, 2], ppad])
    cpad = jnp.full((_CPAD2 - C,), 50.0, jnp.float32)
    cx = jnp.concatenate([cpos[:, 0], cpad])
    cy = jnp.concatenate([cpos[:, 1], cpad])
    cz = jnp.concatenate([cpos[:, 2], cpad])
    mesh = plsc.VectorSubcoreMesh(
        core_axis_name="c", subcore_axis_name="s", num_cores=2, num_subcores=16
    )
    oidx, ocnt = pl.kernel(
        _bq_body,
        out_type=[
            jax.ShapeDtypeStruct((_CPAD2 * K,), jnp.int32),
            jax.ShapeDtypeStruct((_CPAD2,), jnp.int32),
        ],
        mesh=mesh,
        compiler_params=pltpu.CompilerParams(needs_layout_passes=False),
        scratch_types=[
            pltpu.VMEM((_PCH,), jnp.float32),
            pltpu.VMEM((_PCH,), jnp.float32),
            pltpu.VMEM((_PCH,), jnp.float32),
            pltpu.VMEM((_CPER,), jnp.float32),
            pltpu.VMEM((_CPER,), jnp.float32),
            pltpu.VMEM((_CPER,), jnp.float32),
            pltpu.VMEM((_GC * _CAPC,), jnp.float32),
            pltpu.VMEM((_GC * _CAPC,), jnp.int32),
            pltpu.VMEM((16,), jnp.int32),
            pltpu.VMEM((256,), jnp.int32),
            pltpu.VMEM((256,), jnp.int32),
            pltpu.VMEM((_NBB,), jnp.float32),
            pltpu.VMEM((_NBB,), jnp.int32),
            pltpu.VMEM((K,), jnp.int32),
            pltpu.VMEM((_CPER,), jnp.int32),
        ],
    )(px, py, pz, cx, cy, cz)
    return oidx, ocnt


_MROWS = 50176
_E = _CPAD2 * K
_EB = 784
_EPER = _E // _NSC


def _m1_body(xp_ref, w_ref, o_ref):
    o_ref[...] = jnp.dot(xp_ref[...], w_ref[...],
                         preferred_element_type=jnp.float32)


def _xw_pallas(x, pos, cpos_pad, W1):
    xp = jnp.concatenate([x, pos], axis=1)
    xp = jnp.pad(xp, ((0, _MROWS - N), (0, 5)))
    cp = jnp.pad(cpos_pad, ((0, 0), (D, 5)))
    allrows = jnp.concatenate([xp, cp], axis=0)
    w = jnp.pad(W1, ((0, 5), (0, 0)))
    nb = (_MROWS + _CPAD2) // 256
    out = pl.pallas_call(
        _m1_body,
        grid=(nb,),
        in_specs=[
            pl.BlockSpec((256, 136), lambda i: (i, 0)),
            pl.BlockSpec((136, H1), lambda i: (0, 0)),
        ],
        out_specs=pl.BlockSpec((256, H1), lambda i: (i, 0)),
        out_shape=jax.ShapeDtypeStruct((_MROWS + _CPAD2, H1), jnp.float32),
    )(allrows, w)
    return out[:_MROWS], out[_MROWS:]


def _gather_body(xw_h, idx_h, g_h, idxb, rowsb, sem):
    wid = lax.axis_index("s") * 2 + lax.axis_index("c")
    base = wid * _EPER

    def step(t, _):
        r0 = base + t * 128
        pltpu.sync_copy(idx_h.at[pl.ds(r0, 128)], idxb)
        pltpu.async_copy(xw_h.at[idxb], rowsb, sem).wait()
        pltpu.sync_copy(rowsb, g_h.at[pl.ds(r0, 128)])
        return 0

    lax.fori_loop(0, _EPER // 128, step, 0)


def _gather_sc(xw, nbr_flat):
    mesh = plsc.VectorSubcoreMesh(
        core_axis_name="c", subcore_axis_name="s", num_cores=2, num_subcores=16
    )
    return pl.kernel(
        _gather_body,
        out_type=[jax.ShapeDtypeStruct((_E, H1), jnp.float32)],
        mesh=mesh,
        compiler_params=pltpu.CompilerParams(needs_layout_passes=False),
        scratch_types=[
            pltpu.VMEM((128,), jnp.int32),
            pltpu.VMEM((128, H1), jnp.float32),
            pltpu.SemaphoreType.DMA,
        ],
    )(xw, nbr_flat)[0]


def _stats1_body(g_ref, cw_ref, cnt_ref, o_ref):
    i = pl.program_id(0)

    @pl.when(i == 0)
    def _():
        o_ref[...] = jnp.zeros((8, H1), jnp.float32)

    h1 = g_ref[...] - cw_ref[...][:, None, :]
    k_io = jax.lax.broadcasted_iota(jnp.int32, (16, K, H1), 1)
    mf = jnp.where(k_io < cnt_ref[...].astype(jnp.int32)[:, None, :], 1.0, 0.0)
    hm = h1 * mf
    s = jnp.sum(hm, axis=(0, 1)).reshape(1, H1)
    sq = jnp.sum(hm * h1, axis=(0, 1)).reshape(1, H1)
    c = jnp.sum(mf, axis=(0, 1)).reshape(1, H1)
    o_ref[pl.ds(0, 1), :] += s
    o_ref[pl.ds(1, 1), :] += sq
    o_ref[pl.ds(2, 1), :] += c


def _layer1_body(g_ref, cw_ref, cnt_ref, st_ref, w2_ref, gb_ref, h2_ref, o_ref):
    i = pl.program_id(0)

    @pl.when(i == 0)
    def _():
        o_ref[...] = jnp.zeros((8, H1), jnp.float32)

    cnt = jnp.maximum(st_ref[pl.ds(2, 1), :], 1.0)
    mean = st_ref[pl.ds(0, 1), :] / cnt
    var = st_ref[pl.ds(1, 1), :] / cnt - mean * mean
    inv = 1.0 / jnp.sqrt(var + EPS)
    h1 = g_ref[...] - cw_ref[...][:, None, :]
    hn = (h1 - mean[None]) * inv[None]
    hr = jnp.maximum(hn * gb_ref[pl.ds(0, 1), :][None] + gb_ref[pl.ds(1, 1), :][None], 0.0)
    h2 = jnp.dot(hr.reshape(16 * K, H1), w2_ref[...],
                 preferred_element_type=jnp.float32).reshape(16, K, H2)
    h2_ref[...] = h2
    k_io = jax.lax.broadcasted_iota(jnp.int32, (16, K, H2), 1)
    mf = jnp.where(k_io < cnt_ref[...].astype(jnp.int32)[:, None, :], 1.0, 0.0)
    hm = h2 * mf
    o_ref[pl.ds(0, 1), :] += jnp.sum(hm, axis=(0, 1)).reshape(1, H2)
    o_ref[pl.ds(1, 1), :] += jnp.sum(hm * h2, axis=(0, 1)).reshape(1, H2)


def _layer2_body(h2_ref, cnt_ref, st_ref, gb_ref, o_ref):
    cnt = jnp.maximum(st_ref[pl.ds(2, 1), :], 1.0)
    mean = st_ref[pl.ds(0, 1), :] / cnt
    var = st_ref[pl.ds(1, 1), :] / cnt - mean * mean
    inv = 1.0 / jnp.sqrt(var + EPS)
    hn = (h2_ref[...] - mean[None]) * inv[None]
    hr = jnp.maximum(hn * gb_ref[pl.ds(0, 1), :][None] + gb_ref[pl.ds(1, 1), :][None], 0.0)
    k_io = jax.lax.broadcasted_iota(jnp.int32, (16, K, H2), 1)
    msk = k_io < cnt_ref[...].astype(jnp.int32)[:, None, :]
    o_ref[...] = jnp.max(jnp.where(msk, hr, -jnp.inf), axis=1)


def _mlp_pallas(x, pos, cpos_pad, nbr_flat, cnt_pad, W1, g1, b1, W2, g2, b2):
    xw, cw = _xw_pallas(x, pos, cpos_pad, W1)
    g = _gather_sc(xw, nbr_flat)
    g3 = g.reshape(_CPAD2, K, H1)
    cntf = jnp.broadcast_to(cnt_pad.astype(jnp.float32)[:, None], (_CPAD2, H1))
    gb1 = jnp.stack([g1, b1])
    gb2 = jnp.stack([g2, b2])
    st1 = pl.pallas_call(
        _stats1_body,
        grid=(_EB,),
        in_specs=[
            pl.BlockSpec((16, K, H1), lambda i: (i, 0, 0)),
            pl.BlockSpec((16, H1), lambda i: (i, 0)),
            pl.BlockSpec((16, H1), lambda i: (i, 0)),
        ],
        out_specs=pl.BlockSpec((8, H1), lambda i: (0, 0)),
        out_shape=jax.ShapeDtypeStruct((8, H1), jnp.float32),
    )(g3, cw, cntf)
    h2, st2p = pl.pallas_call(
        _layer1_body,
        grid=(_EB,),
        in_specs=[
            pl.BlockSpec((16, K, H1), lambda i: (i, 0, 0)),
            pl.BlockSpec((16, H1), lambda i: (i, 0)),
            pl.BlockSpec((16, H1), lambda i: (i, 0)),
            pl.BlockSpec((8, H1), lambda i: (0, 0)),
            pl.BlockSpec((H1, H2), lambda i: (0, 0)),
            pl.BlockSpec((2, H1), lambda i: (0, 0)),
        ],
        out_specs=[
            pl.BlockSpec((16, K, H2), lambda i: (i, 0, 0)),
            pl.BlockSpec((8, H2), lambda i: (0, 0)),
        ],
        out_shape=[
            jax.ShapeDtypeStruct((_CPAD2, K, H2), jnp.float32),
            jax.ShapeDtypeStruct((8, H2), jnp.float32),
        ],
    )(g3, cw, cntf, st1, W2, gb1)
    st2 = jnp.concatenate([st2p[:2], st1[2:3]], axis=0)
    st2 = jnp.pad(st2, ((0, 5), (0, 0)))
    out = pl.pallas_call(
        _layer2_body,
        grid=(_EB,),
        in_specs=[
            pl.BlockSpec((16, K, H2), lambda i: (i, 0, 0)),
            pl.BlockSpec((16, H2), lambda i: (i, 0)),
            pl.BlockSpec((8, H2), lambda i: (0, 0)),
            pl.BlockSpec((2, H2), lambda i: (0, 0)),
        ],
        out_specs=pl.BlockSpec((16, H2), lambda i: (i, 0)),
        out_shape=jax.ShapeDtypeStruct((_CPAD2, H2), jnp.float32),
    )(h2, cntf, st2, gb2)
    return out


def kernel(x, pos, batch, W1, g1, b1, W2, g2, b2):
    idx = _fps_pallas(pos)
    cpos = pos[idx]
    nbr_flat, cnt_pad = _ball_query_flat(pos, cpos)
    cpos_pad = jnp.pad(cpos, ((0, _CPAD2 - C), (0, 0)))
    out = _mlp_pallas(x, pos, cpos_pad, nbr_flat, cnt_pad,
                      W1, g1, b1, W2, g2, b2)
    return out[:C], cpos, batch[idx]


# speedup vs baseline: 1.0327x; 1.0327x over previous
import functools

import jax
import jax.numpy as jnp
from jax import lax
from jax.experimental import pallas as pl
from jax.experimental.pallas import tpu as pltpu
from jax.experimental.pallas import tpu_sc as plsc

N = 50000
D = 128
R = 0.2
K = 64
C = 12500
CHUNK = 250
H1 = 128
H2 = 128
EPS = 1e-5

_G = (N + 1023) // 1024
_NPAD = _G * 1024
_CROWS = (C + 127) // 128
_CPAD = _CROWS * 128
_BIG_I32 = 2**30


def _fps_body(pxyz_ref, out_ref, dists_ref):
    flat3 = (
        jax.lax.broadcasted_iota(jnp.int32, (_G, 8, 128), 0) * 1024
        + jax.lax.broadcasted_iota(jnp.int32, (_G, 8, 128), 1) * 128
        + jax.lax.broadcasted_iota(jnp.int32, (_G, 8, 128), 2)
    )
    dists_ref[...] = jnp.where(flat3 < N, jnp.inf, -jnp.inf).astype(jnp.float32)

    def body(i, carry):
        last, rowv = carry
        g = last // 1024
        s = (last // 128) % 8
        l = last % 128
        tile = pxyz_ref[:, pl.ds(g, 1)]
        s_io = jax.lax.broadcasted_iota(jnp.int32, (1, 8, 128), 1)
        l_io = jax.lax.broadcasted_iota(jnp.int32, (1, 8, 128), 2)
        sel = jnp.logical_and(s_io == s, l_io == l)
        lx = jnp.sum(jnp.where(sel, tile[0], 0.0))
        ly = jnp.sum(jnp.where(sel, tile[1], 0.0))
        lz = jnp.sum(jnp.where(sel, tile[2], 0.0))
        vmax = jnp.full((1, 8, 128), -jnp.inf, jnp.float32)
        vpl = jnp.zeros((1, 8, 128), jnp.int32)
        for gg in range(_G):
            dx = pxyz_ref[0, pl.ds(gg, 1)] - lx
            dy = pxyz_ref[1, pl.ds(gg, 1)] - ly
            dz = pxyz_ref[2, pl.ds(gg, 1)] - lz
            d = (dx * dx + dz * dz) + dy * dy
            nd = jnp.minimum(dists_ref[pl.ds(gg, 1)], d)
            dists_ref[pl.ds(gg, 1)] = nd
            upd = nd > vmax
            vmax = jnp.where(upd, nd, vmax)
            vpl = jnp.where(upd, gg, vpl)
        m = jnp.max(vmax)
        slot_flat = s_io * 128 + l_io
        nxt = jnp.min(jnp.where(vmax == m, vpl * 1024 + slot_flat, _BIG_I32))
        lane = i % 128
        rowv = jnp.where(lax.broadcasted_iota(jnp.int32, (1, 128), 1) == lane,
                         nxt, rowv)

        @pl.when(lane == 127)
        def _():
            out_ref[pl.ds(i // 128, 1), :] = rowv

        return nxt, rowv

    _, rowv = jax.lax.fori_loop(
        1, C, body, (jnp.int32(0), jnp.zeros((1, 128), jnp.int32)))
    out_ref[pl.ds(_CROWS - 1, 1), :] = rowv


def _fps_pallas(pos):
    pad = jnp.full((_NPAD - N, 3), 30.0, jnp.float32)
    pxyz = jnp.concatenate([pos, pad], axis=0).T.reshape(3, _G, 8, 128)
    out = pl.pallas_call(
        _fps_body,
        out_shape=jax.ShapeDtypeStruct((_CROWS, 128), jnp.int32),
        scratch_shapes=[pltpu.VMEM((_G, 8, 128), jnp.float32)],
    )(pxyz)
    return out.reshape(_CPAD)[:C]


_NSC = 32
_CPER = 392
_CPAD2 = _NSC * _CPER
_PCH = 12544
_NPAD3 = 4 * _PCH
_GC = 8
_CAPC = 4096
_NBB = 1024
import numpy as _np

_RR = float(_np.float32(0.2) * _np.float32(0.2))
_INF = float("inf")


def _ilist(x):
    return [x]


def _bq_body(px_h, py_h, pz_h, cx_h, cy_h, cz_h, oidx_h, ocnt_h,
             pxb, pyb, pzb, cbx, cby, cbz, cand_d, cand_i, cntb,
             hist, cum, bbd, bbi, orow, ocb):
    wid = lax.axis_index("s") * 2 + lax.axis_index("c")
    cbase = wid * _CPER
    pltpu.sync_copy(cx_h.at[pl.ds(cbase, _CPER)], cbx)
    pltpu.sync_copy(cy_h.at[pl.ds(cbase, _CPER)], cby)
    pltpu.sync_copy(cz_h.at[pl.ds(cbase, _CPER)], cbz)
    l16 = lax.iota(jnp.int32, 16)
    lane0 = l16 == 0
    zero16 = jnp.zeros((16,), jnp.int32)

    def splat_load(ref, i):
        return plsc.load_gather(ref, _ilist(zero16 + i))

    def group_loop(g, _):
        cntb[pl.ds(0, 16)] = zero16

        def chunk_loop(ch, _):
            off = ch * _PCH
            pltpu.sync_copy(px_h.at[pl.ds(off, _PCH)], pxb)
            pltpu.sync_copy(py_h.at[pl.ds(off, _PCH)], pyb)
            pltpu.sync_copy(pz_h.at[pl.ds(off, _PCH)], pzb)

            def cent_loop(cc, _):
                cl = g * _GC + cc
                cxv = splat_load(cbx, cl)
                cyv = splat_load(cby, cl)
                czv = splat_load(cbz, cl)
                cnt0 = splat_load(cntb, cc)

                def vec_loop(j, cnt):
                    ds_ = []
                    ms_ = []
                    pcs = []
                    for u in range(16):
                        b = j * 256 + u * 16
                        dx = pxb[pl.ds(b, 16)] - cxv
                        dy = pyb[pl.ds(b, 16)] - cyv
                        dz = pzb[pl.ds(b, 16)] - czv
                        d = (dx * dx + dz * dz) + dy * dy
                        m = d <= _RR
                        ds_.append(d)
                        ms_.append(m)
                        pcs.append(plsc.cumsum(m.astype(jnp.int32)))
                    for u in range(16):
                        b = j * 256 + u * 16
                        pos = jnp.minimum(cnt + pcs[u] - 1, _CAPC - 1) + cc * _CAPC
                        pid = off + b + l16
                        plsc.store_scatter(cand_d, _ilist(pos), ds_[u], mask=ms_[u])
                        plsc.store_scatter(cand_i, _ilist(pos), pid, mask=ms_[u])
                        cnt = cnt + plsc.all_reduce_population_count(ms_[u])
                    return cnt

                cnt = lax.fori_loop(0, _PCH // 256, vec_loop, cnt0)
                plsc.store_scatter(cntb, _ilist(zero16 + cc), cnt,
                                   mask=lane0)
                return 0

            lax.fori_loop(0, _GC, cent_loop, 0)
            return 0

        lax.fori_loop(0, 4, chunk_loop, 0)

        def select_loop(cc, _):
            cl = g * _GC + cc
            total = jnp.minimum(jnp.max(splat_load(cntb, cc)), _CAPC)
            nv = (total + 15) // 16
            cb = cc * _CAPC

            def hclr(k, _):
                hist[pl.ds(k * 16, 16)] = zero16
                return 0

            lax.fori_loop(0, 16, hclr, 0)

            def hsweep(j, _):
                b = cb + j * 16
                lm = (j * 16 + l16) < total
                d = jnp.where(lm, cand_d[pl.ds(b, 16)], 0.0)
                bk = jnp.minimum((d * 6400.0).astype(jnp.int32), 255)
                plsc.addupdate_scatter(hist, _ilist(bk),
                                       jnp.where(lm, 1, 0), mask=lm)
                return 0

            lax.fori_loop(0, nv, hsweep, 0)

            def csum(k, carry):
                s = plsc.cumsum(hist[pl.ds(k * 16, 16)]) + carry
                cum[pl.ds(k * 16, 16)] = s
                return jnp.max(s)

            lax.fori_loop(0, 16, csum, jnp.int32(0))

            def bstar_loop(k, carry):
                bs, lo = carry
                cv = cum[pl.ds(k * 16, 16)]
                bio = k * 16 + l16
                bs = jnp.minimum(bs, jnp.min(jnp.where(cv >= K, bio, 1024)))
                lo = jnp.maximum(lo, jnp.max(jnp.where(cv < K, cv, 0)))
                return bs, lo

            bstar, lo = lax.fori_loop(0, 16, bstar_loop,
                                      (jnp.int32(1024), jnp.int32(0)))
            need = jnp.where(bstar < 1024, K - lo, 0)

            def oclr(k, _):
                orow[pl.ds(k * 16, 16)] = zero16
                return 0

            lax.fori_loop(0, K // 16, oclr, 0)

            def ssweep(j, carry):
                osel, bbc = carry
                b = cb + j * 16
                lm = (j * 16 + l16) < total
                d = jnp.where(lm, cand_d[pl.ds(b, 16)], _INF)
                pid = cand_i[pl.ds(b, 16)]
                bk = jnp.minimum((d * 6400.0).astype(jnp.int32), 255)
                sm = lm & (bk < bstar)
                pc = plsc.cumsum(sm.astype(jnp.int32))
                pos = jnp.minimum(osel + pc - 1, K - 1)
                plsc.store_scatter(orow, _ilist(pos), pid, mask=sm)
                osel = osel + plsc.all_reduce_population_count(sm)
                bm = lm & (bk == bstar)
                pc2 = plsc.cumsum(bm.astype(jnp.int32))
                pos2 = jnp.minimum(bbc + pc2 - 1, _NBB - 1)
                plsc.store_scatter(bbd, _ilist(pos2), d, mask=bm)
                plsc.store_scatter(bbi, _ilist(pos2), pid, mask=bm)
                bbc = bbc + plsc.all_reduce_population_count(bm)
                return osel, bbc

            osel, bbc = lax.fori_loop(0, nv, ssweep, (zero16, zero16))
            nb = jnp.minimum(jnp.max(bbc), _NBB)
            nbv = (nb + 15) // 16

            def extract(_, osel):
                def minp(j, ms):
                    lm = (j * 16 + l16) < nb
                    v = jnp.where(lm, bbd[pl.ds(j * 16, 16)], _INF)
                    return jnp.minimum(ms, jnp.min(v))

                ms = lax.fori_loop(0, nbv, minp, jnp.float32(_INF))

                def idxp(j, ix):
                    lm = (j * 16 + l16) < nb
                    v = bbd[pl.ds(j * 16, 16)]
                    pid = bbi[pl.ds(j * 16, 16)]
                    sel = lm & (v == ms)
                    return jnp.minimum(ix, jnp.min(jnp.where(sel, pid, _BIG_I32)))

                ix = lax.fori_loop(0, nbv, idxp, jnp.int32(_BIG_I32))

                def markp(j, _):
                    b = j * 16
                    v = bbd[pl.ds(b, 16)]
                    pid = bbi[pl.ds(b, 16)]
                    hit = (v == ms) & (pid == ix)
                    bbd[pl.ds(b, 16)] = jnp.where(hit, _INF, v)
                    return 0

                lax.fori_loop(0, nbv, markp, 0)
                plsc.store_scatter(orow, _ilist(jnp.minimum(osel, K - 1)),
                                   zero16 + ix, mask=lane0)
                return osel + 1

            osel = lax.fori_loop(0, need, extract, osel)
            pltpu.sync_copy(orow, oidx_h.at[pl.ds((cbase + cl) * K, K)])
            plsc.store_scatter(ocb, _ilist(zero16 + cl),
                               jnp.minimum(osel, K), mask=lane0)
            return 0

        lax.fori_loop(0, _GC, select_loop, 0)
        return 0

    lax.fori_loop(0, _CPER // _GC, group_loop, 0)
    pltpu.sync_copy(ocb, ocnt_h.at[pl.ds(cbase, _CPER)])


def _ball_query_flat(pos, cpos):
    ppad = jnp.full((_NPAD3 - N,), 30.0, jnp.float32)
    px = jnp.concatenate([pos[:, 0], ppad])
    py = jnp.concatenate([pos[:, 1], ppad])
    pz = jnp.concatenate([pos[:, 2], ppad])
    cpad = jnp.full((_CPAD2 - C,), 50.0, jnp.float32)
    cx = jnp.concatenate([cpos[:, 0], cpad])
    cy = jnp.concatenate([cpos[:, 1], cpad])
    cz = jnp.concatenate([cpos[:, 2], cpad])
    mesh = plsc.VectorSubcoreMesh(
        core_axis_name="c", subcore_axis_name="s", num_cores=2, num_subcores=16
    )
    oidx, ocnt = pl.kernel(
        _bq_body,
        out_type=[
            jax.ShapeDtypeStruct((_CPAD2 * K,), jnp.int32),
            jax.ShapeDtypeStruct((_CPAD2,), jnp.int32),
        ],
        mesh=mesh,
        compiler_params=pltpu.CompilerParams(needs_layout_passes=False),
        scratch_types=[
            pltpu.VMEM((_PCH,), jnp.float32),
            pltpu.VMEM((_PCH,), jnp.float32),
            pltpu.VMEM((_PCH,), jnp.float32),
            pltpu.VMEM((_CPER,), jnp.float32),
            pltpu.VMEM((_CPER,), jnp.float32),
            pltpu.VMEM((_CPER,), jnp.float32),
            pltpu.VMEM((_GC * _CAPC,), jnp.float32),
            pltpu.VMEM((_GC * _CAPC,), jnp.int32),
            pltpu.VMEM((16,), jnp.int32),
            pltpu.VMEM((256,), jnp.int32),
            pltpu.VMEM((256,), jnp.int32),
            pltpu.VMEM((_NBB,), jnp.float32),
            pltpu.VMEM((_NBB,), jnp.int32),
            pltpu.VMEM((K,), jnp.int32),
            pltpu.VMEM((_CPER,), jnp.int32),
        ],
    )(px, py, pz, cx, cy, cz)
    return oidx, ocnt


_MROWS = 50176
_E = _CPAD2 * K
_EB = 784
_EPER = _E // _NSC


def _m1_body(xp_ref, w_ref, o_ref):
    o_ref[...] = jnp.dot(xp_ref[...], w_ref[...],
                         preferred_element_type=jnp.float32)


def _xw_pallas(x, pos, cpos_pad, W1):
    xp = jnp.concatenate([x, pos], axis=1)
    xp = jnp.pad(xp, ((0, _MROWS - N), (0, 5)))
    cp = jnp.pad(cpos_pad, ((0, 0), (D, 5)))
    allrows = jnp.concatenate([xp, cp], axis=0)
    w = jnp.pad(W1, ((0, 5), (0, 0)))
    nb = (_MROWS + _CPAD2) // 256
    out = pl.pallas_call(
        _m1_body,
        grid=(nb,),
        in_specs=[
            pl.BlockSpec((256, 136), lambda i: (i, 0)),
            pl.BlockSpec((136, H1), lambda i: (0, 0)),
        ],
        out_specs=pl.BlockSpec((256, H1), lambda i: (i, 0)),
        out_shape=jax.ShapeDtypeStruct((_MROWS + _CPAD2, H1), jnp.float32),
    )(allrows, w)
    return out[:_MROWS], out[_MROWS:]


def _gather_body(xw_h, idx_h, g_h, idxb, rowsb, sem):
    wid = lax.axis_index("s") * 2 + lax.axis_index("c")
    base = wid * _EPER

    def step(t, _):
        r0 = base + t * 128
        pltpu.sync_copy(idx_h.at[pl.ds(r0, 128)], idxb)
        pltpu.async_copy(xw_h.at[idxb], rowsb, sem).wait()
        pltpu.sync_copy(rowsb, g_h.at[pl.ds(r0, 128)])
        return 0

    lax.fori_loop(0, _EPER // 128, step, 0)


def _gather_sc(xw, nbr_flat):
    mesh = plsc.VectorSubcoreMesh(
        core_axis_name="c", subcore_axis_name="s", num_cores=2, num_subcores=16
    )
    return pl.kernel(
        _gather_body,
        out_type=[jax.ShapeDtypeStruct((_E, H1), jnp.float32)],
        mesh=mesh,
        compiler_params=pltpu.CompilerParams(needs_layout_passes=False),
        scratch_types=[
            pltpu.VMEM((128,), jnp.int32),
            pltpu.VMEM((128, H1), jnp.float32),
            pltpu.SemaphoreType.DMA,
        ],
    )(xw, nbr_flat)[0]


def _stats1_body(g_ref, cw_ref, cnt_ref, o_ref):
    i = pl.program_id(0)

    @pl.when(i == 0)
    def _():
        o_ref[...] = jnp.zeros((8, H1), jnp.float32)

    h1 = g_ref[...] - cw_ref[...][:, None, :]
    k_io = jax.lax.broadcasted_iota(jnp.int32, (16, K, H1), 1)
    mf = jnp.where(k_io < cnt_ref[...].astype(jnp.int32)[:, None, :], 1.0, 0.0)
    hm = h1 * mf
    s = jnp.sum(hm, axis=(0, 1)).reshape(1, H1)
    sq = jnp.sum(hm * h1, axis=(0, 1)).reshape(1, H1)
    c = jnp.sum(mf, axis=(0, 1)).reshape(1, H1)
    o_ref[pl.ds(0, 1), :] += s
    o_ref[pl.ds(1, 1), :] += sq
    o_ref[pl.ds(2, 1), :] += c


def _layer1_body(g_ref, cw_ref, cnt_ref, st_ref, w2_ref, gb_ref, h2_ref, o_ref):
    i = pl.program_id(0)

    @pl.when(i == 0)
    def _():
        o_ref[...] = jnp.zeros((8, H1), jnp.float32)

    cnt = jnp.maximum(st_ref[pl.ds(2, 1), :], 1.0)
    mean = st_ref[pl.ds(0, 1), :] / cnt
    var = st_ref[pl.ds(1, 1), :] / cnt - mean * mean
    inv = 1.0 / jnp.sqrt(var + EPS)
    h1 = g_ref[...] - cw_ref[...][:, None, :]
    hn = (h1 - mean[None]) * inv[None]
    hr = jnp.maximum(hn * gb_ref[pl.ds(0, 1), :][None] + gb_ref[pl.ds(1, 1), :][None], 0.0)
    h2 = jnp.dot(hr.reshape(16 * K, H1), w2_ref[...],
                 preferred_element_type=jnp.float32).reshape(16, K, H2)
    h2_ref[...] = h2
    k_io = jax.lax.broadcasted_iota(jnp.int32, (16, K, H2), 1)
    mf = jnp.where(k_io < cnt_ref[...].astype(jnp.int32)[:, None, :], 1.0, 0.0)
    hm = h2 * mf
    o_ref[pl.ds(0, 1), :] += jnp.sum(hm, axis=(0, 1)).reshape(1, H2)
    o_ref[pl.ds(1, 1), :] += jnp.sum(hm * h2, axis=(0, 1)).reshape(1, H2)


def _layer2_body(h2_ref, cnt_ref, st_ref, gb_ref, o_ref):
    cnt = jnp.maximum(st_ref[pl.ds(2, 1), :], 1.0)
    mean = st_ref[pl.ds(0, 1), :] / cnt
    var = st_ref[pl.ds(1, 1), :] / cnt - mean * mean
    inv = 1.0 / jnp.sqrt(var + EPS)
    hn = (h2_ref[...] - mean[None]) * inv[None]
    hr = jnp.maximum(hn * gb_ref[pl.ds(0, 1), :][None] + gb_ref[pl.ds(1, 1), :][None], 0.0)
    k_io = jax.lax.broadcasted_iota(jnp.int32, (16, K, H2), 1)
    msk = k_io < cnt_ref[...].astype(jnp.int32)[:, None, :]
    o_ref[...] = jnp.max(jnp.where(msk, hr, -jnp.inf), axis=1)


def _mlp_pallas(x, pos, cpos_pad, nbr_flat, cnt_pad, W1, g1, b1, W2, g2, b2):
    xw, cw = _xw_pallas(x, pos, cpos_pad, W1)
    g = _gather_sc(xw, nbr_flat)
    g3 = g.reshape(_CPAD2, K, H1)
    cntf = jnp.broadcast_to(cnt_pad.astype(jnp.float32)[:, None], (_CPAD2, H1))
    gb1 = jnp.stack([g1, b1])
    gb2 = jnp.stack([g2, b2])
    st1 = pl.pallas_call(
        _stats1_body,
        grid=(_EB,),
        in_specs=[
            pl.BlockSpec((16, K, H1), lambda i: (i, 0, 0)),
            pl.BlockSpec((16, H1), lambda i: (i, 0)),
            pl.BlockSpec((16, H1), lambda i: (i, 0)),
        ],
        out_specs=pl.BlockSpec((8, H1), lambda i: (0, 0)),
        out_shape=jax.ShapeDtypeStruct((8, H1), jnp.float32),
    )(g3, cw, cntf)
    h2, st2p = pl.pallas_call(
        _layer1_body,
        grid=(_EB,),
        in_specs=[
            pl.BlockSpec((16, K, H1), lambda i: (i, 0, 0)),
            pl.BlockSpec((16, H1), lambda i: (i, 0)),
            pl.BlockSpec((16, H1), lambda i: (i, 0)),
            pl.BlockSpec((8, H1), lambda i: (0, 0)),
            pl.BlockSpec((H1, H2), lambda i: (0, 0)),
            pl.BlockSpec((2, H1), lambda i: (0, 0)),
        ],
        out_specs=[
            pl.BlockSpec((16, K, H2), lambda i: (i, 0, 0)),
            pl.BlockSpec((8, H2), lambda i: (0, 0)),
        ],
        out_shape=[
            jax.ShapeDtypeStruct((_CPAD2, K, H2), jnp.float32),
            jax.ShapeDtypeStruct((8, H2), jnp.float32),
        ],
    )(g3, cw, cntf, st1, W2, gb1)
    st2 = jnp.concatenate([st2p[:2], st1[2:3]], axis=0)
    st2 = jnp.pad(st2, ((0, 5), (0, 0)))
    out = pl.pallas_call(
        _layer2_body,
        grid=(_EB,),
        in_specs=[
            pl.BlockSpec((16, K, H2), lambda i: (i, 0, 0)),
            pl.BlockSpec((16, H2), lambda i: (i, 0)),
            pl.BlockSpec((8, H2), lambda i: (0, 0)),
            pl.BlockSpec((2, H2), lambda i: (0, 0)),
        ],
        out_specs=pl.BlockSpec((16, H2), lambda i: (i, 0)),
        out_shape=jax.ShapeDtypeStruct((_CPAD2, H2), jnp.float32),
    )(h2, cntf, st2, gb2)
    return out


def kernel(x, pos, batch, W1, g1, b1, W2, g2, b2):
    idx = _fps_pallas(pos)
    cpos = pos[idx]
    nbr_flat, cnt_pad = _ball_query_flat(pos, cpos)
    cpos_pad = jnp.pad(cpos, ((0, _CPAD2 - C), (0, 0)))
    out = _mlp_pallas(x, pos, cpos_pad, nbr_flat, cnt_pad,
                      W1, g1, b1, W2, g2, b2)
    return out[:C], cpos, batch[idx]
